# baseline (speedup 1.0000x reference)
import jax
import jax.numpy as jnp
from jax import lax
from jax.experimental import pallas as pl
from jax.experimental.pallas import tpu as pltpu
from jax.experimental.pallas import tpu_sc as plsc

VOCAB = 8192
N_TOK = 8192
SUB = 8
LANE = VOCAB // SUB

NC, NS = 2, 16
NW = NC * NS
TPW = N_TOK // NW
CH = 8
NCHUNK = TPW // CH
VHALF = VOCAB // 2
NG = TPW // 16


def _sc_mesh():
    return plsc.VectorSubcoreMesh(
        core_axis_name="c", subcore_axis_name="s", num_cores=NC, num_subcores=NS
    )


def _sc_gather_body(table, idx, out, idx_v, buf0, buf1, sem0, sem1):
    wid = lax.axis_index("s") * NC + lax.axis_index("c")
    base = wid * TPW
    pltpu.sync_copy(idx.at[pl.ds(base, TPW)], idx_v)

    def src(c, h):
        return table.at[idx_v.at[pl.ds(c * CH, CH)], pl.ds(h * VHALF, VHALF)]

    def dst(c, h):
        return out.at[pl.ds(base + c * CH, CH), pl.ds(h * VHALF, VHALF)]

    pltpu.async_copy(src(0, 0), buf0, sem0)

    def step(c, carry):
        pltpu.make_async_copy(src(c, 0), buf0, sem0).wait()
        pltpu.async_copy(src(c, 1), buf1, sem1)
        pltpu.sync_copy(buf0, dst(c, 0))
        pltpu.make_async_copy(src(c, 1), buf1, sem1).wait()

        @pl.when(c + 1 < NCHUNK)
        def _():
            pltpu.async_copy(src(c + 1, 0), buf0, sem0)

        pltpu.sync_copy(buf1, dst(c, 1))
        return carry

    lax.fori_loop(0, NCHUNK, step, 0)


def _sc_gather(embds, flat_idx):
    f = pl.kernel(
        _sc_gather_body,
        out_type=jax.ShapeDtypeStruct((N_TOK, VOCAB), jnp.float32),
        mesh=_sc_mesh(),
        scratch_types=[
            pltpu.VMEM((TPW,), jnp.int32),
            pltpu.VMEM((CH, VHALF), jnp.float32),
            pltpu.VMEM((CH, VHALF), jnp.float32),
            pltpu.SemaphoreType.DMA,
            pltpu.SemaphoreType.DMA,
        ],
    )
    return f(embds, flat_idx)


KL = 32
GRID_L = VOCAB // KL


def _lse_body(x_ref, lse_ref):
    X = x_ref[...]
    m = jnp.max(X, axis=1, keepdims=True)
    s = jnp.sum(jnp.exp(X - m), axis=1, keepdims=True)
    lse_ref[...] = (m + jnp.log(s)).reshape(1, 1, KL)


def _tc_lse(embds):
    return pl.pallas_call(
        _lse_body,
        grid=(GRID_L,),
        in_specs=[pl.BlockSpec((KL, VOCAB), lambda i: (i, 0))],
        out_specs=pl.BlockSpec((1, 1, KL), lambda i: (i, 0, 0)),
        out_shape=jax.ShapeDtypeStruct((GRID_L, 1, KL), jnp.float32),
    )(embds)


def _sc_tval_body(table, idx, tgt, out_tval, idx_v, tgt_v, tval_v, tiles, sem):
    wid = lax.axis_index("s") * NC + lax.axis_index("c")
    base = wid * TPW
    pltpu.sync_copy(idx.at[pl.ds(base, TPW)], idx_v)
    pltpu.sync_copy(tgt.at[pl.ds(base, TPW)], tgt_v)

    lane16 = lax.iota(jnp.int32, 16)

    def group(g, carry):
        idx16 = idx_v[pl.ds(g * 16, 16)]
        t16 = tgt_v[pl.ds(g * 16, 16)]
        for j in range(16):
            row_al = (idx16[j] // 8) * 8
            col_al = (t16[j] // 128) * 128
            pltpu.make_async_copy(
                table.at[pl.ds(row_al, 8), pl.ds(col_al, 128)],
                tiles.at[j],
                sem,
            ).start()
        for j in range(16):
            pltpu.make_async_copy(
                table.at[pl.ds(0, 8), pl.ds(0, 128)], tiles.at[j], sem
            ).wait()

        one = jnp.int32(1)
        for j in range(16):
            r_j = idx16[j] % 8
            c_j = t16[j] % 128
            c16 = (c_j // 16) * 16
            l_j = c_j % 16
            lmask = (one - jnp.minimum(one, jnp.abs(lane16 - l_j))).astype(
                jnp.float32
            )
            acc = (lane16 * 0).astype(jnp.float32)
            for r in range(8):
                vec = tiles[j, r, pl.ds(c16, 16)]
                rz = (one - jnp.minimum(one, jnp.abs(r_j - r))).astype(
                    jnp.float32
                )
                acc = acc + vec * lmask * rz
            tval_v[g * 16 + j] = acc
        return carry

    lax.fori_loop(0, NG, group, 0)
    pltpu.sync_copy(tval_v, out_tval.at[pl.ds(base, TPW)])


def _sc_tval(embds, flat_idx, flat_tgt):
    f = pl.kernel(
        _sc_tval_body,
        out_type=jax.ShapeDtypeStruct((N_TOK, 16), jnp.float32),
        mesh=_sc_mesh(),
        scratch_types=[
            pltpu.VMEM((TPW,), jnp.int32),
            pltpu.VMEM((TPW,), jnp.int32),
            pltpu.VMEM((TPW, 16), jnp.float32),
            pltpu.VMEM((16, 8, 128), jnp.float32),
            pltpu.SemaphoreType.DMA,
        ],
    )
    return f(embds, flat_idx, flat_tgt)


def _sc_select_body(lse, idx, out_sel, idx_v, sel_v, sem0):
    wid = lax.axis_index("s") * NC + lax.axis_index("c")
    base = wid * TPW
    pltpu.sync_copy(idx.at[pl.ds(base, TPW)], idx_v)
    pltpu.async_copy(lse.at[idx_v], sel_v, sem0)
    pltpu.make_async_copy(lse.at[idx_v], sel_v, sem0).wait()
    pltpu.sync_copy(sel_v, out_sel.at[pl.ds(base, TPW)])


def _sc_select(lse_flat, flat_idx):
    f = pl.kernel(
        _sc_select_body,
        out_type=jax.ShapeDtypeStruct((N_TOK,), jnp.float32),
        mesh=_sc_mesh(),
        scratch_types=[
            pltpu.VMEM((TPW,), jnp.int32),
            pltpu.VMEM((TPW,), jnp.float32),
            pltpu.SemaphoreType.DMA,
        ],
    )
    return f(lse_flat, flat_idx)


def _mean_body(sel_ref, tval_ref, loss_ref):
    loss_ref[0, 0] = (jnp.sum(sel_ref[...]) - jnp.sum(tval_ref[...])) * (
        1.0 / N_TOK
    )


def _tc_mean(sel, tval16):
    loss = pl.pallas_call(
        _mean_body,
        in_specs=[
            pl.BlockSpec((1, SUB, LANE), lambda: (0, 0, 0)),
            pl.BlockSpec((N_TOK, 16), lambda: (0, 0)),
        ],
        out_specs=pl.BlockSpec(memory_space=pltpu.SMEM),
        out_shape=jax.ShapeDtypeStruct((1, 1), jnp.float32),
    )(sel.reshape(1, SUB, LANE), tval16)
    return loss[0, 0]


@jax.jit
def _run(flat_idx, flat_tgt, embds):
    logits = _sc_gather(embds, flat_idx)
    lse = _tc_lse(embds)
    tval = _sc_tval(embds, flat_idx, flat_tgt)
    sel = _sc_select(lse.reshape(-1), flat_idx)
    loss = _tc_mean(sel, tval)
    return logits, loss


def kernel(inputs, targets, embds):
    Bq, Tq = inputs.shape
    flat_idx = inputs.reshape(-1).astype(jnp.int32)
    flat_tgt = targets.reshape(-1).astype(jnp.int32)
    logits, loss = _run(flat_idx, flat_tgt, embds)
    return logits.reshape(Bq, Tq, VOCAB), loss


# speedup vs baseline: 1.2898x; 1.2898x over previous
import jax
import jax.numpy as jnp
from jax import lax
from jax.experimental import pallas as pl
from jax.experimental.pallas import tpu as pltpu
from jax.experimental.pallas import tpu_sc as plsc

VOCAB = 8192
N_TOK = 8192
SUB = 8
LANE = VOCAB // SUB

NC, NS = 2, 16
NW = NC * NS
TPW = N_TOK // NW
CH = 8
NCHUNK = TPW // CH
NPAIRS = NCHUNK // 2
VHALF = VOCAB // 2


def _sc_mesh():
    return plsc.VectorSubcoreMesh(
        core_axis_name="c", subcore_axis_name="s", num_cores=NC, num_subcores=NS
    )


def _sc_gather_body(table, idx, tgt, out, out_tval,
                    idx_v, tgt_v, tot_v, buf0, buf1, sem0, sem1):
    wid = lax.axis_index("s") * NC + lax.axis_index("c")
    base = wid * TPW
    pltpu.sync_copy(idx.at[pl.ds(base, TPW)], idx_v)
    pltpu.sync_copy(tgt.at[pl.ds(base, TPW)], tgt_v)

    lane16 = lax.iota(jnp.int32, 16)
    one = jnp.int32(1)

    def src(c, h):
        return table.at[idx_v.at[pl.ds(c * CH, CH)], pl.ds(h * VHALF, VHALF)]

    def dst(c, h):
        return out.at[pl.ds(base + c * CH, CH), pl.ds(h * VHALF, VHALF)]

    def extract(tt, parity):
        part = (lane16 * 0).astype(jnp.float32)
        for r in range(CH):
            t_j = tt[parity * CH + r]
            d = t_j // VHALF
            cc0 = jnp.clip(t_j, 0, VHALF - 1)
            cc1 = jnp.clip(t_j - VHALF, 0, VHALF - 1)
            for h, buf, cc in ((0, buf0, cc0), (1, buf1, cc1)):
                inh = (one - jnp.abs(d - h)).astype(jnp.float32)
                c16 = (cc // 16) * 16
                l_j = cc % 16
                lmask = (
                    one - jnp.minimum(one, jnp.abs(lane16 - l_j))
                ).astype(jnp.float32)
                vec = buf[r, pl.ds(c16, 16)]
                part = part + vec * lmask * inh
        return part

    pltpu.async_copy(src(0, 0), buf0, sem0)

    def pairstep(c2, tot):
        tt = tgt_v[pl.ds(c2 * 16, 16)]
        for parity in range(2):
            c = c2 * 2 + parity
            pltpu.make_async_copy(src(c, 0), buf0, sem0).wait()
            pltpu.async_copy(src(c, 1), buf1, sem1)
            pltpu.sync_copy(buf0, dst(c, 0))
            pltpu.make_async_copy(src(c, 1), buf1, sem1).wait()
            tot = tot + extract(tt, parity)

            @pl.when(c + 1 < NCHUNK)
            def _():
                pltpu.async_copy(src(c + 1, 0), buf0, sem0)

            pltpu.sync_copy(buf1, dst(c, 1))
        return tot

    tot = lax.fori_loop(
        0, NPAIRS, pairstep, (lane16 * 0).astype(jnp.float32)
    )
    tot_v[0] = tot
    pltpu.sync_copy(tot_v, out_tval.at[pl.ds(wid, 1)])


def _sc_gather(embds, flat_idx, flat_tgt):
    f = pl.kernel(
        _sc_gather_body,
        out_type=(
            jax.ShapeDtypeStruct((N_TOK, VOCAB), jnp.float32),
            jax.ShapeDtypeStruct((NW, 16), jnp.float32),
        ),
        mesh=_sc_mesh(),
        scratch_types=[
            pltpu.VMEM((TPW,), jnp.int32),
            pltpu.VMEM((TPW,), jnp.int32),
            pltpu.VMEM((1, 16), jnp.float32),
            pltpu.VMEM((CH, VHALF), jnp.float32),
            pltpu.VMEM((CH, VHALF), jnp.float32),
            pltpu.SemaphoreType.DMA,
            pltpu.SemaphoreType.DMA,
        ],
    )
    return f(embds, flat_idx, flat_tgt)


KL = 32
GRID_L = VOCAB // (2 * KL)


def _lse_body(x0_ref, x1_ref, lse_ref):
    def one_blk(X):
        m = jnp.max(X, axis=1, keepdims=True)
        s = jnp.sum(jnp.exp(X - m), axis=1, keepdims=True)
        return (m + jnp.log(s)).reshape(1, 1, KL)

    lse_ref[0, 0] = one_blk(x0_ref[...])[0, 0]
    lse_ref[0, 1] = one_blk(x1_ref[...])[0, 0]


def _tc_lse(embds):
    return pl.pallas_call(
        _lse_body,
        grid=(GRID_L,),
        in_specs=[
            pl.BlockSpec((KL, VOCAB), lambda i: (2 * i, 0)),
            pl.BlockSpec((KL, VOCAB), lambda i: (2 * i + 1, 0)),
        ],
        out_specs=pl.BlockSpec((1, 2, KL), lambda i: (i, 0, 0)),
        out_shape=jax.ShapeDtypeStruct((GRID_L, 2, KL), jnp.float32),
    )(embds, embds)


def _sc_select_body(lse, idx, out_sel, idx_v, sel_v, sem0):
    wid = lax.axis_index("s") * NC + lax.axis_index("c")
    base = wid * TPW
    pltpu.sync_copy(idx.at[pl.ds(base, TPW)], idx_v)
    pltpu.async_copy(lse.at[idx_v], sel_v, sem0)
    pltpu.make_async_copy(lse.at[idx_v], sel_v, sem0).wait()
    pltpu.sync_copy(sel_v, out_sel.at[pl.ds(base, TPW)])


def _sc_select(lse_flat, flat_idx):
    f = pl.kernel(
        _sc_select_body,
        out_type=jax.ShapeDtypeStruct((N_TOK,), jnp.float32),
        mesh=_sc_mesh(),
        scratch_types=[
            pltpu.VMEM((TPW,), jnp.int32),
            pltpu.VMEM((TPW,), jnp.float32),
            pltpu.SemaphoreType.DMA,
        ],
    )
    return f(lse_flat, flat_idx)


def _mean_body(sel_ref, tval_ref, loss_ref):
    loss_ref[0, 0] = (jnp.sum(sel_ref[...]) - jnp.sum(tval_ref[...])) * (
        1.0 / N_TOK
    )


def _tc_mean(sel, tval):
    loss = pl.pallas_call(
        _mean_body,
        in_specs=[
            pl.BlockSpec((1, SUB, LANE), lambda: (0, 0, 0)),
            pl.BlockSpec((NW, 16), lambda: (0, 0)),
        ],
        out_specs=pl.BlockSpec(memory_space=pltpu.SMEM),
        out_shape=jax.ShapeDtypeStruct((1, 1), jnp.float32),
    )(sel.reshape(1, SUB, LANE), tval)
    return loss[0, 0]


@jax.jit
def _run(flat_idx, flat_tgt, embds):
    logits, tval = _sc_gather(embds, flat_idx, flat_tgt)
    lse = _tc_lse(embds)
    sel = _sc_select(lse.reshape(-1), flat_idx)
    loss = _tc_mean(sel, tval)
    return logits, loss


def kernel(inputs, targets, embds):
    Bq, Tq = inputs.shape
    flat_idx = inputs.reshape(-1).astype(jnp.int32)
    flat_tgt = targets.reshape(-1).astype(jnp.int32)
    logits, loss = _run(flat_idx, flat_tgt, embds)
    return logits.reshape(Bq, Tq, VOCAB), loss


# speedup vs baseline: 1.3293x; 1.0307x over previous
import jax
import jax.numpy as jnp
from jax import lax
from jax.experimental import pallas as pl
from jax.experimental.pallas import tpu as pltpu
from jax.experimental.pallas import tpu_sc as plsc

VOCAB = 8192
N_TOK = 8192
SUB = 8
LANE = VOCAB // SUB

NC, NS = 2, 16
NW = NC * NS
TPW = N_TOK // NW
CH = 8
NCHUNK = TPW // CH
NPAIRS = NCHUNK // 2
VHALF = VOCAB // 2


def _sc_mesh():
    return plsc.VectorSubcoreMesh(
        core_axis_name="c", subcore_axis_name="s", num_cores=NC, num_subcores=NS
    )


def _sc_gather_body(table, idx, tgt, out, out_tval,
                    idx_v, tgt_v, tot_v, buf0, buf1, sem0, sem1):
    wid = lax.axis_index("s") * NC + lax.axis_index("c")
    base = wid * TPW
    pltpu.sync_copy(idx.at[pl.ds(base, TPW)], idx_v)
    pltpu.sync_copy(tgt.at[pl.ds(base, TPW)], tgt_v)

    lane16 = lax.iota(jnp.int32, 16)
    one = jnp.int32(1)

    def src(c, h):
        return table.at[idx_v.at[pl.ds(c * CH, CH)], pl.ds(h * VHALF, VHALF)]

    def dst(c, h):
        return out.at[pl.ds(base + c * CH, CH), pl.ds(h * VHALF, VHALF)]

    def extract(tt, parity):
        part = (lane16 * 0).astype(jnp.float32)
        for r in range(CH):
            t_j = tt[parity * CH + r]
            d = t_j // VHALF
            cc0 = jnp.clip(t_j, 0, VHALF - 1)
            cc1 = jnp.clip(t_j - VHALF, 0, VHALF - 1)
            for h, buf, cc in ((0, buf0, cc0), (1, buf1, cc1)):
                inh = (one - jnp.abs(d - h)).astype(jnp.float32)
                c16 = (cc // 16) * 16
                l_j = cc % 16
                lmask = (
                    one - jnp.minimum(one, jnp.abs(lane16 - l_j))
                ).astype(jnp.float32)
                vec = buf[r, pl.ds(c16, 16)]
                part = part + vec * lmask * inh
        return part

    pltpu.async_copy(src(0, 0), buf0, sem0)

    def pairstep(c2, tot):
        tt = tgt_v[pl.ds(c2 * 16, 16)]
        for parity in range(2):
            c = c2 * 2 + parity
            pltpu.make_async_copy(src(c, 0), buf0, sem0).wait()
            pltpu.async_copy(src(c, 1), buf1, sem1)
            pltpu.sync_copy(buf0, dst(c, 0))
            pltpu.make_async_copy(src(c, 1), buf1, sem1).wait()
            tot = tot + extract(tt, parity)

            @pl.when(c + 1 < NCHUNK)
            def _():
                pltpu.async_copy(src(c + 1, 0), buf0, sem0)

            pltpu.sync_copy(buf1, dst(c, 1))
        return tot

    tot = lax.fori_loop(
        0, NPAIRS, pairstep, (lane16 * 0).astype(jnp.float32)
    )
    tot_v[0] = tot
    pltpu.sync_copy(tot_v, out_tval.at[pl.ds(wid, 1)])


def _sc_gather(embds, flat_idx, flat_tgt):
    f = pl.kernel(
        _sc_gather_body,
        out_type=(
            jax.ShapeDtypeStruct((N_TOK, VOCAB), jnp.float32),
            jax.ShapeDtypeStruct((NW, 16), jnp.float32),
        ),
        mesh=_sc_mesh(),
        scratch_types=[
            pltpu.VMEM((TPW,), jnp.int32),
            pltpu.VMEM((TPW,), jnp.int32),
            pltpu.VMEM((1, 16), jnp.float32),
            pltpu.VMEM((CH, VHALF), jnp.float32),
            pltpu.VMEM((CH, VHALF), jnp.float32),
            pltpu.SemaphoreType.DMA,
            pltpu.SemaphoreType.DMA,
        ],
    )
    return f(embds, flat_idx, flat_tgt)


KL = 32
GRID_L = VOCAB // (2 * KL)


def _lse_body(x0_ref, x1_ref, cnt_ref, acc_ref):
    i = pl.program_id(0)

    @pl.when(i == 0)
    def _():
        acc_ref[0, 0] = 0.0

    def one_blk(X):
        m = jnp.max(X, axis=1, keepdims=True)
        s = jnp.sum(jnp.exp(X - m), axis=1, keepdims=True)
        return (m + jnp.log(s)).reshape(1, 1, KL)

    w0 = jnp.sum(one_blk(x0_ref[...])[0, 0] * cnt_ref[0, 0])
    w1 = jnp.sum(one_blk(x1_ref[...])[0, 0] * cnt_ref[0, 1])
    acc_ref[0, 0] += w0 + w1


def _tc_lse_weighted(embds, counts3):
    return pl.pallas_call(
        _lse_body,
        grid=(GRID_L,),
        in_specs=[
            pl.BlockSpec((KL, VOCAB), lambda i: (2 * i, 0)),
            pl.BlockSpec((KL, VOCAB), lambda i: (2 * i + 1, 0)),
            pl.BlockSpec((1, 2, KL), lambda i: (i, 0, 0)),
        ],
        out_specs=pl.BlockSpec(
            (1, 1), lambda i: (0, 0), memory_space=pltpu.SMEM
        ),
        out_shape=jax.ShapeDtypeStruct((1, 1), jnp.float32),
    )(embds, embds, counts3)


def _mean_body(wsum_ref, tval_ref, loss_ref):
    loss_ref[0, 0] = (wsum_ref[0, 0] - jnp.sum(tval_ref[...])) * (1.0 / N_TOK)


def _tc_mean(wsum, tval):
    loss = pl.pallas_call(
        _mean_body,
        in_specs=[
            pl.BlockSpec((1, 1), lambda: (0, 0), memory_space=pltpu.SMEM),
            pl.BlockSpec((NW, 16), lambda: (0, 0)),
        ],
        out_specs=pl.BlockSpec(memory_space=pltpu.SMEM),
        out_shape=jax.ShapeDtypeStruct((1, 1), jnp.float32),
    )(wsum, tval)
    return loss[0, 0]


@jax.jit
def _run(flat_idx, flat_tgt, embds):
    logits, tval = _sc_gather(embds, flat_idx, flat_tgt)
    counts = jnp.zeros((VOCAB,), jnp.float32).at[flat_idx].add(1.0)
    wsum = _tc_lse_weighted(embds, counts.reshape(GRID_L, 2, KL))
    loss = _tc_mean(wsum, tval)
    return logits, loss


def kernel(inputs, targets, embds):
    Bq, Tq = inputs.shape
    flat_idx = inputs.reshape(-1).astype(jnp.int32)
    flat_tgt = targets.reshape(-1).astype(jnp.int32)
    logits, loss = _run(flat_idx, flat_tgt, embds)
    return logits.reshape(Bq, Tq, VOCAB), loss
